# initial kernel scaffold (unmeasured)
import functools

import jax
import jax.numpy as jnp
from jax import lax
from jax.experimental import pallas as pl
from jax.experimental.pallas import tpu as pltpu

N_DEV = 32
N_ROUNDS = 5
B, SQ, D = 1, 512, 1024
HQ, DH = 8, 128
SCALE = 0.08838834764831843


def kernel(x, Wq, Wo, K_ext, V_ext):
    def body(x_ref, wq_ref, wo_ref, k_ref, v_ref, out_ref,
             acc_ref, stats_ref, acc_rx, stats_rx,
             acc_send_sems, acc_recv_sems, st_send_sems, st_recv_sems):
        me = lax.axis_index("i")

        barrier_sem = pltpu.get_barrier_semaphore()
        for r in range(N_ROUNDS):
            partner = jnp.bitwise_xor(me, 1 << r)
            pl.semaphore_signal(barrier_sem, inc=1, device_id=(partner,),
                                device_id_type=pl.DeviceIdType.MESH)
        pl.semaphore_wait(barrier_sem, N_ROUNDS)

        xb = x_ref[0].astype(jnp.bfloat16)
        q = lax.dot(xb, wq_ref[...].astype(jnp.bfloat16),
                    preferred_element_type=jnp.float32) * SCALE
        for h in range(HQ):
            qh = q[:, h * DH:(h + 1) * DH].astype(jnp.bfloat16)
            kh = k_ref[0, :, h, :].astype(jnp.bfloat16)
            vh = v_ref[0, :, h, :].astype(jnp.bfloat16)
            s = lax.dot_general(qh, kh, (((1,), (1,)), ((), ())),
                                preferred_element_type=jnp.float32)
            mh = jnp.max(s, axis=1)
            p = jnp.exp(s - mh[:, None])
            lh = jnp.sum(p, axis=1)
            acc_ref[h] = lax.dot(p.astype(jnp.bfloat16), vh,
                                 preferred_element_type=jnp.float32)
            stats_ref[0, h] = mh
            stats_ref[1, h] = lh

        for r in range(N_ROUNDS):
            partner = jnp.bitwise_xor(me, 1 << r)
            acc_rdma = pltpu.make_async_remote_copy(
                src_ref=acc_ref, dst_ref=acc_rx.at[r],
                send_sem=acc_send_sems.at[r], recv_sem=acc_recv_sems.at[r],
                device_id=(partner,), device_id_type=pl.DeviceIdType.MESH)
            st_rdma = pltpu.make_async_remote_copy(
                src_ref=stats_ref, dst_ref=stats_rx.at[r],
                send_sem=st_send_sems.at[r], recv_sem=st_recv_sems.at[r],
                device_id=(partner,), device_id_type=pl.DeviceIdType.MESH)
            acc_rdma.start()
            st_rdma.start()
            acc_rdma.wait()
            st_rdma.wait()

            m_a = stats_ref[0]
            l_a = stats_ref[1]
            m_b = stats_rx[r, 0]
            l_b = stats_rx[r, 1]
            m_n = jnp.maximum(m_a, m_b)
            aa = jnp.exp(m_a - m_n)
            ab = jnp.exp(m_b - m_n)
            stats_ref[0] = m_n
            stats_ref[1] = l_a * aa + l_b * ab
            acc_ref[...] = (acc_ref[...] * aa[:, :, None]
                            + acc_rx[r] * ab[:, :, None])

        l_f = stats_ref[1]
        on = acc_ref[...] / l_f[:, :, None]
        o2 = jnp.concatenate([on[h] for h in range(HQ)], axis=1)
        out_ref[0] = lax.dot(o2.astype(jnp.bfloat16),
                             wo_ref[...].astype(jnp.bfloat16),
                             preferred_element_type=jnp.float32)

        @functools.partial(pl.run_scoped,
                           exit_sem=pltpu.SemaphoreType.REGULAR)
        def _(exit_sem):
            for r in range(N_ROUNDS):
                partner = jnp.bitwise_xor(me, 1 << r)
                pl.semaphore_signal(exit_sem, inc=1, device_id=(partner,),
                                    device_id_type=pl.DeviceIdType.MESH)
            pl.semaphore_wait(exit_sem, N_ROUNDS)

    return pl.pallas_call(
        body,
        out_shape=jax.ShapeDtypeStruct((B, SQ, D), jnp.float32),
        in_specs=[pl.BlockSpec(memory_space=pltpu.VMEM)] * 5,
        out_specs=pl.BlockSpec(memory_space=pltpu.VMEM),
        scratch_shapes=[
            pltpu.VMEM((HQ, SQ, DH), jnp.float32),
            pltpu.VMEM((2, HQ, SQ), jnp.float32),
            pltpu.VMEM((N_ROUNDS, HQ, SQ, DH), jnp.float32),
            pltpu.VMEM((N_ROUNDS, 2, HQ, SQ), jnp.float32),
            pltpu.SemaphoreType.DMA((N_ROUNDS,)),
            pltpu.SemaphoreType.DMA((N_ROUNDS,)),
            pltpu.SemaphoreType.DMA((N_ROUNDS,)),
            pltpu.SemaphoreType.DMA((N_ROUNDS,)),
        ],
        compiler_params=pltpu.CompilerParams(collective_id=0),
    )(x, Wq, Wo, K_ext, V_ext)


# baseline (device time: 227910 ns/iter reference)
import functools

import jax
import jax.numpy as jnp
from jax import lax
from jax.experimental import pallas as pl
from jax.experimental.pallas import tpu as pltpu

N_DEV = 32
N_ROUNDS = 5
B, SQ, D = 1, 512, 1024
HQ, DH = 8, 128
SCALE = 0.08838834764831843


def kernel(x, Wq, Wo, K_ext, V_ext):
    def body(x_ref, wq_ref, wo_ref, k_ref, v_ref, out_ref,
             acc_ref, stats_ref, acc_rx, stats_rx,
             acc_send_sems, acc_recv_sems, st_send_sems, st_recv_sems):
        me = lax.axis_index("i")

        barrier_sem = pltpu.get_barrier_semaphore()
        for r in range(N_ROUNDS):
            partner = jnp.bitwise_xor(me, 1 << r)
            pl.semaphore_signal(barrier_sem, inc=1, device_id=(partner,),
                                device_id_type=pl.DeviceIdType.MESH)
        pl.semaphore_wait(barrier_sem, N_ROUNDS)

        xb = x_ref[0].astype(jnp.bfloat16)
        q = lax.dot(xb, wq_ref[...].astype(jnp.bfloat16),
                    preferred_element_type=jnp.float32) * SCALE
        for h in range(HQ):
            qh = q[:, h * DH:(h + 1) * DH].astype(jnp.bfloat16)
            kh = k_ref[0, :, h, :].astype(jnp.bfloat16)
            vh = v_ref[0, :, h, :].astype(jnp.bfloat16)
            s = lax.dot_general(qh, kh, (((1,), (1,)), ((), ())),
                                preferred_element_type=jnp.float32)
            mh = jnp.max(s, axis=1)
            p = jnp.exp(s - mh[:, None])
            lh = jnp.sum(p, axis=1)
            acc_ref[h] = lax.dot(p.astype(jnp.bfloat16), vh,
                                 preferred_element_type=jnp.float32)
            stats_ref[0, h] = mh
            stats_ref[1, h] = lh

        for r in range(N_ROUNDS):
            partner = jnp.bitwise_xor(me, 1 << r)
            acc_rdma = pltpu.make_async_remote_copy(
                src_ref=acc_ref, dst_ref=acc_rx.at[r],
                send_sem=acc_send_sems.at[r], recv_sem=acc_recv_sems.at[r],
                device_id=(partner,), device_id_type=pl.DeviceIdType.MESH)
            st_rdma = pltpu.make_async_remote_copy(
                src_ref=stats_ref, dst_ref=stats_rx.at[r],
                send_sem=st_send_sems.at[r], recv_sem=st_recv_sems.at[r],
                device_id=(partner,), device_id_type=pl.DeviceIdType.MESH)
            acc_rdma.start()
            st_rdma.start()
            acc_rdma.wait()
            st_rdma.wait()

            m_a = stats_ref[0]
            l_a = stats_ref[1]
            m_b = stats_rx[r, 0]
            l_b = stats_rx[r, 1]
            m_n = jnp.maximum(m_a, m_b)
            aa = jnp.exp(m_a - m_n)
            ab = jnp.exp(m_b - m_n)
            stats_ref[0] = m_n
            stats_ref[1] = l_a * aa + l_b * ab
            acc_ref[...] = (acc_ref[...] * aa[:, :, None]
                            + acc_rx[r] * ab[:, :, None])

        l_f = stats_ref[1]
        on = acc_ref[...] / l_f[:, :, None]
        o2 = jnp.concatenate([on[h] for h in range(HQ)], axis=1)
        out_ref[0] = lax.dot(o2.astype(jnp.bfloat16),
                             wo_ref[...].astype(jnp.bfloat16),
                             preferred_element_type=jnp.float32)

        @functools.partial(pl.run_scoped,
                           exit_sem=pltpu.SemaphoreType.REGULAR)
        def _(exit_sem):
            for r in range(N_ROUNDS):
                partner = jnp.bitwise_xor(me, 1 << r)
                pl.semaphore_signal(exit_sem, inc=1, device_id=(partner,),
                                    device_id_type=pl.DeviceIdType.MESH)
            pl.semaphore_wait(exit_sem, N_ROUNDS)

    return pl.pallas_call(
        body,
        out_shape=jax.ShapeDtypeStruct((B, SQ, D), jnp.float32),
        in_specs=[pl.BlockSpec(memory_space=pltpu.VMEM)] * 5,
        out_specs=pl.BlockSpec(memory_space=pltpu.VMEM),
        scratch_shapes=[
            pltpu.VMEM((HQ, SQ, DH), jnp.float32),
            pltpu.VMEM((2, HQ, SQ), jnp.float32),
            pltpu.VMEM((N_ROUNDS, HQ, SQ, DH), jnp.float32),
            pltpu.VMEM((N_ROUNDS, 2, HQ, SQ), jnp.float32),
            pltpu.SemaphoreType.DMA((N_ROUNDS,)),
            pltpu.SemaphoreType.DMA((N_ROUNDS,)),
            pltpu.SemaphoreType.DMA((N_ROUNDS,)),
            pltpu.SemaphoreType.DMA((N_ROUNDS,)),
        ],
        compiler_params=pltpu.CompilerParams(
            collective_id=0, vmem_limit_bytes=100 * 1024 * 1024),
    )(x, Wq, Wo, K_ext, V_ext)


# device time: 111787 ns/iter; 2.0388x vs baseline; 2.0388x over previous
import functools

import jax
import jax.numpy as jnp
from jax import lax
from jax.experimental import pallas as pl
from jax.experimental.pallas import tpu as pltpu

N_DEV = 32
N_ROUNDS = 5
B, SQ, D = 1, 512, 1024
HQ, DH = 8, 128
OWN = SQ // N_DEV
SCALE = 0.08838834764831843
RXOFF = [0, 256, 384, 448, 480]


def kernel(x, Wq, Wo, K_ext, V_ext):
    def body(x_ref, wq_ref, wo_ref, k_ref, v_ref, out_ref,
             acc_ref, acc_tx, stats_ref, acc_rx, stats_rx, out_g,
             rs_a_send, rs_a_recv, rs_s_send, rs_s_recv, ag_send, ag_recv):
        me = lax.axis_index("i")

        barrier_sem = pltpu.get_barrier_semaphore()
        for r in range(N_ROUNDS):
            partner = jnp.bitwise_xor(me, 1 << r)
            pl.semaphore_signal(barrier_sem, inc=1, device_id=(partner,),
                                device_id_type=pl.DeviceIdType.MESH)
        pl.semaphore_wait(barrier_sem, N_ROUNDS)

        xb = x_ref[0].astype(jnp.bfloat16)
        q = lax.dot(xb, wq_ref[...].astype(jnp.bfloat16),
                    preferred_element_type=jnp.float32) * SCALE
        for h in range(HQ):
            qh = q[:, h * DH:(h + 1) * DH].astype(jnp.bfloat16)
            kh = k_ref[0, :, h, :].astype(jnp.bfloat16)
            vh = v_ref[0, :, h, :].astype(jnp.bfloat16)
            s = lax.dot_general(qh, kh, (((1,), (1,)), ((), ())),
                                preferred_element_type=jnp.float32)
            mh = jnp.max(s, axis=1, keepdims=True)
            p = jnp.exp(s - mh)
            lh = jnp.sum(p, axis=1, keepdims=True)
            acch = lax.dot(p.astype(jnp.bfloat16), vh,
                           preferred_element_type=jnp.float32)
            acc_ref[:, h * DH:(h + 1) * DH] = acch
            acc_tx[:, h * DH:(h + 1) * DH] = acch.astype(jnp.bfloat16)
            stats_ref[0, :, h:h + 1] = mh
            stats_ref[1, :, h:h + 1] = lh

        cur_base = jnp.int32(0)
        for t in range(N_ROUNDS):
            b = N_ROUNDS - 1 - t
            half = SQ >> (t + 1)
            partner = jnp.bitwise_xor(me, 1 << b)
            mybit = jnp.right_shift(me, b) & 1
            keep_base = cur_base + mybit * half
            send_base = cur_base + (1 - mybit) * half

            a_rdma = pltpu.make_async_remote_copy(
                src_ref=acc_tx.at[pl.ds(send_base, half), :],
                dst_ref=acc_rx.at[pl.ds(RXOFF[t], half), :],
                send_sem=rs_a_send.at[t], recv_sem=rs_a_recv.at[t],
                device_id=(partner,), device_id_type=pl.DeviceIdType.MESH)
            s_rdma = pltpu.make_async_remote_copy(
                src_ref=stats_ref.at[:, pl.ds(send_base, half), :],
                dst_ref=stats_rx.at[:, pl.ds(RXOFF[t], half), :],
                send_sem=rs_s_send.at[t], recv_sem=rs_s_recv.at[t],
                device_id=(partner,), device_id_type=pl.DeviceIdType.MESH)
            a_rdma.start()
            s_rdma.start()
            a_rdma.wait()
            s_rdma.wait()

            m_a = stats_ref[0, pl.ds(keep_base, half), :]
            l_a = stats_ref[1, pl.ds(keep_base, half), :]
            m_b = stats_rx[0, pl.ds(RXOFF[t], half), :]
            l_b = stats_rx[1, pl.ds(RXOFF[t], half), :]
            m_n = jnp.maximum(m_a, m_b)
            aa = jnp.exp(m_a - m_n)
            ab = jnp.exp(m_b - m_n)
            stats_ref[0, pl.ds(keep_base, half), :] = m_n
            stats_ref[1, pl.ds(keep_base, half), :] = l_a * aa + l_b * ab
            for h in range(HQ):
                cs = slice(h * DH, (h + 1) * DH)
                a_h = acc_ref[pl.ds(keep_base, half), cs]
                b_h = acc_rx[pl.ds(RXOFF[t], half), cs].astype(jnp.float32)
                merged = a_h * aa[:, h:h + 1] + b_h * ab[:, h:h + 1]
                acc_ref[pl.ds(keep_base, half), cs] = merged
                if t < N_ROUNDS - 1:
                    acc_tx[pl.ds(keep_base, half), cs] = (
                        merged.astype(jnp.bfloat16))
            cur_base = keep_base

        my16 = acc_ref[pl.ds(cur_base, OWN), :]
        l16 = stats_ref[1, pl.ds(cur_base, OWN), :]
        on = jnp.concatenate(
            [my16[:, h * DH:(h + 1) * DH] / l16[:, h:h + 1] for h in range(HQ)],
            axis=1)
        o = lax.dot(on.astype(jnp.bfloat16), wo_ref[...].astype(jnp.bfloat16),
                    preferred_element_type=jnp.float32)
        out_g[pl.ds(cur_base, OWN), :] = o.astype(jnp.bfloat16)

        for t in range(N_ROUNDS):
            gsize = OWN << t
            partner = jnp.bitwise_xor(me, 1 << t)
            gbase = (jnp.right_shift(me, t) << t) * OWN
            g_rdma = pltpu.make_async_remote_copy(
                src_ref=out_g.at[pl.ds(gbase, gsize), :],
                dst_ref=out_g.at[pl.ds(gbase, gsize), :],
                send_sem=ag_send.at[t], recv_sem=ag_recv.at[t],
                device_id=(partner,), device_id_type=pl.DeviceIdType.MESH)
            g_rdma.start()
            g_rdma.wait()

        out_ref[0] = out_g[...].astype(jnp.float32)

        @functools.partial(pl.run_scoped,
                           exit_sem=pltpu.SemaphoreType.REGULAR)
        def _(exit_sem):
            for r in range(N_ROUNDS):
                partner = jnp.bitwise_xor(me, 1 << r)
                pl.semaphore_signal(exit_sem, inc=1, device_id=(partner,),
                                    device_id_type=pl.DeviceIdType.MESH)
            pl.semaphore_wait(exit_sem, N_ROUNDS)

    return pl.pallas_call(
        body,
        out_shape=jax.ShapeDtypeStruct((B, SQ, D), jnp.float32),
        in_specs=[pl.BlockSpec(memory_space=pltpu.VMEM)] * 5,
        out_specs=pl.BlockSpec(memory_space=pltpu.VMEM),
        scratch_shapes=[
            pltpu.VMEM((SQ, D), jnp.float32),
            pltpu.VMEM((SQ, D), jnp.bfloat16),
            pltpu.VMEM((2, SQ, HQ), jnp.float32),
            pltpu.VMEM((SQ, D), jnp.bfloat16),
            pltpu.VMEM((2, SQ, HQ), jnp.float32),
            pltpu.VMEM((SQ, D), jnp.bfloat16),
            pltpu.SemaphoreType.DMA((N_ROUNDS,)),
            pltpu.SemaphoreType.DMA((N_ROUNDS,)),
            pltpu.SemaphoreType.DMA((N_ROUNDS,)),
            pltpu.SemaphoreType.DMA((N_ROUNDS,)),
            pltpu.SemaphoreType.DMA((N_ROUNDS,)),
            pltpu.SemaphoreType.DMA((N_ROUNDS,)),
        ],
        compiler_params=pltpu.CompilerParams(
            collective_id=0, vmem_limit_bytes=100 * 1024 * 1024),
    )(x, Wq, Wo, K_ext, V_ext)
